# baseline (speedup 1.0000x reference)
import math

import jax
import jax.numpy as jnp
from jax.experimental import pallas as pl
from jax.experimental.pallas import tpu as pltpu

N = 512
H = 64
NR = 60
NB = 4
NM = 32
CUTOFF = 5.0
TI = 32
NCH = N // TI
E = TI * N
KG = NR + TI

_HIGHEST = jax.lax.Precision.HIGHEST


def _silu_from_half(r):
    return r + r * jnp.tanh(r)


def _body(an_ref, posr_ref, pos_ref, batc_ref, emb_ref, *rest):
    f32 = jnp.float32
    wrefs = rest[:9 * NB]
    (ow1_ref, ob1_ref, ow2_ref, ob2_ref, y_ref,
     g_ref, d_ref, m_ref) = rest[9 * NB:]

    an = jnp.clip(an_ref[...], 0, 99)
    onehot = (jax.lax.broadcasted_iota(jnp.int32, (100, N), 0) == an).astype(f32)
    xT = jnp.dot(emb_ref[...], onehot, preferred_element_type=f32)

    pos = pos_ref[...]
    posr = posr_ref[...]

    width = CUTOFF / NR
    inv = 1.0 / (2.0 * width * width)
    centers_s = (jax.lax.broadcasted_iota(jnp.int32, (NR, 1), 0).astype(f32)
                 * (CUTOFF / (NR - 1)))

    dsq = ((posr[:, 0:1] - pos[0:1, :]) ** 2
           + (posr[:, 1:2] - pos[1:2, :]) ** 2
           + (posr[:, 2:3] - pos[2:3, :]) ** 2)
    d = jnp.sqrt(dsq)
    fr_row = jax.lax.broadcasted_iota(jnp.int32, (N, N), 0)
    fr_col = jax.lax.broadcasted_iota(jnp.int32, (N, N), 1)
    keep = (fr_row != fr_col) & (d < CUTOFF)
    maskf = keep.astype(f32)
    d_ref[...] = jnp.where(keep, d, 1e4)
    m_ref[...] = maskf
    cnt = jnp.dot(jnp.ones((1, N), f32), maskf, preferred_element_type=f32)

    g_ref[NR:KG, :] = jnp.zeros((TI, E), f32)

    ei_row = jax.lax.broadcasted_iota(jnp.int32, (N, TI), 0)
    ei_col = jax.lax.broadcasted_iota(jnp.int32, (N, TI), 1)

    for b in range(NB):
        w1x, w1r, b1, w2, b2, u1, ub1, u2, ub2 = wrefs[9 * b:9 * (b + 1)]
        xw1h = jnp.dot(w1x[...] * 0.5, xT, preferred_element_type=f32)
        w1rh, b1h = w1r[...] * 0.5, b1[...] * 0.5
        u1h, ub1h = u1[...] * 0.5, ub1[...] * 0.5

        def chunk(c, hsum, xw1h=xw1h, w1rh=w1rh, b1h=b1h):
            for t in range(TI):
                drow = d_ref[pl.ds(c * TI + t, 1), :]
                mrow = m_ref[pl.ds(c * TI + t, 1), :]
                y = drow - centers_s
                g_ref[0:NR, t * N:(t + 1) * N] = jnp.exp((y * y) * (-inv))
                g_ref[NR + t:NR + t + 1, t * N:(t + 1) * N] = mrow
            ec = (ei_row == c * TI + ei_col).astype(f32)
            xc = jnp.dot(xw1h, ec, preferred_element_type=f32) + b1h
            wcat = jnp.concatenate([w1rh, xc], axis=1)
            r = jnp.dot(wcat, g_ref[...], preferred_element_type=f32)
            for t in range(TI):
                hsum = hsum + _silu_from_half(r[:, t * N:(t + 1) * N])
            return hsum

        hsum = jax.lax.fori_loop(0, NCH, chunk, jnp.zeros((H, N), f32),
                                 unroll=2)

        aggr = jnp.dot(w2[...], hsum, preferred_element_type=f32) + b2[...] * cnt
        u = jnp.concatenate([xT, aggr], axis=0)
        hu = _silu_from_half(
            jnp.dot(u1h, u, preferred_element_type=f32) + ub1h)
        xT = xT + jnp.dot(u2[...], hu, preferred_element_type=f32) + ub2[...]

    sel = (batc_ref[...] == jax.lax.broadcasted_iota(jnp.int32, (1, NM), 1)).astype(f32)
    mol = jnp.dot(xT, sel, preferred_element_type=f32)
    cntm = jnp.sum(sel, axis=0, keepdims=True)
    mol = mol / jnp.clip(cntm, 1.0, None)
    ho = _silu_from_half(
        jnp.dot(ow1_ref[...] * 0.5, mol, preferred_element_type=f32)
        + ob1_ref[...] * 0.5)
    y_ref[...] = jnp.dot(ow2_ref[...], ho, preferred_element_type=f32) + ob2_ref[...]


def kernel(atomic_numbers, positions, batch, emb, blocks, out_w1, out_b1, out_w2, out_b2):
    f32 = jnp.float32
    anT = jnp.asarray(atomic_numbers, jnp.int32).reshape(1, N)
    posr = jnp.asarray(positions, f32)
    posT = posr.T
    batc = jnp.asarray(batch, jnp.int32).reshape(N, 1)
    embT = jnp.asarray(emb, f32).T
    wflat = []
    for blk in blocks:
        wflat += [
            blk['msg_w1'][:H].T, blk['msg_w1'][H:].T, blk['msg_b1'].reshape(H, 1),
            blk['msg_w2'].T, blk['msg_b2'].reshape(H, 1),
            blk['upd_w1'].T, blk['upd_b1'].reshape(H, 1),
            blk['upd_w2'].T, blk['upd_b2'].reshape(H, 1),
        ]
    yT = pl.pallas_call(
        _body,
        out_shape=jax.ShapeDtypeStruct((1, NM), f32),
        scratch_shapes=[pltpu.VMEM((KG, E), f32),
                        pltpu.VMEM((N, N), f32),
                        pltpu.VMEM((N, N), f32)],
    )(anT, posr, posT, batc, embT, *wflat,
      out_w1.T, out_b1.reshape(H // 2, 1), out_w2.T, out_b2.reshape(1, 1))
    return yT.reshape(NM, 1)


# speedup vs baseline: 1.0311x; 1.0311x over previous
import math

import jax
import jax.numpy as jnp
from jax.experimental import pallas as pl
from jax.experimental.pallas import tpu as pltpu

N = 512
H = 64
NR = 60
NB = 4
NM = 32
CUTOFF = 5.0
TI = 32
NCH = N // TI
E = TI * N
KG = NR + TI

_HIGHEST = jax.lax.Precision.HIGHEST


def _silu_from_half(r):
    return r + r * jnp.tanh(r)


def _body(an_ref, posr_ref, pos_ref, batc_ref, emb_ref, *rest):
    f32 = jnp.float32
    wrefs = rest[:9 * NB]
    (ow1_ref, ob1_ref, ow2_ref, ob2_ref, y_ref,
     g_ref, d_ref, m_ref) = rest[9 * NB:]

    an = jnp.clip(an_ref[...], 0, 99)
    onehot = (jax.lax.broadcasted_iota(jnp.int32, (100, N), 0) == an).astype(f32)
    xT = jnp.dot(emb_ref[...], onehot, preferred_element_type=f32)

    pos = pos_ref[...]
    posr = posr_ref[...]

    width = CUTOFF / NR
    inv = 1.0 / (2.0 * width * width)
    scale = math.sqrt(inv * math.log2(math.e))
    centers_s = (jax.lax.broadcasted_iota(jnp.int32, (NR, 1), 0).astype(f32)
                 * (CUTOFF / (NR - 1) * scale))

    cross = jax.lax.dot_general(posr, pos, (((1,), (0,)), ((), ())),
                                precision=_HIGHEST,
                                preferred_element_type=f32)
    p2row = jnp.sum(posr * posr, axis=1, keepdims=True)
    p2col = jnp.sum(pos * pos, axis=0, keepdims=True)
    dsq = jnp.maximum(p2row + p2col - 2.0 * cross, 0.0)
    fr_row = jax.lax.broadcasted_iota(jnp.int32, (N, N), 0)
    fr_col = jax.lax.broadcasted_iota(jnp.int32, (N, N), 1)
    keep = (fr_row != fr_col) & (dsq < CUTOFF * CUTOFF)
    maskf = keep.astype(f32)
    d_ref[...] = jnp.where(keep, jnp.sqrt(dsq) * scale, 1e4)
    m_ref[...] = maskf
    cnt = jnp.dot(jnp.ones((1, N), f32), maskf, preferred_element_type=f32)

    g_ref[NR:KG, :] = jnp.zeros((TI, E), f32)

    ei_row = jax.lax.broadcasted_iota(jnp.int32, (N, TI), 0)
    ei_col = jax.lax.broadcasted_iota(jnp.int32, (N, TI), 1)

    for b in range(NB):
        w1x, w1r, b1, w2, b2, u1, ub1, u2, ub2 = wrefs[9 * b:9 * (b + 1)]
        xw1h = jnp.dot(w1x[...] * 0.5, xT, preferred_element_type=f32)
        w1rh, b1h = w1r[...] * 0.5, b1[...] * 0.5
        u1h, ub1h = u1[...] * 0.5, ub1[...] * 0.5

        def chunk(c, hsum, xw1h=xw1h, w1rh=w1rh, b1h=b1h):
            for t in range(TI):
                drow = d_ref[pl.ds(c * TI + t, 1), :]
                mrow = m_ref[pl.ds(c * TI + t, 1), :]
                y = drow - centers_s
                g_ref[0:NR, t * N:(t + 1) * N] = jnp.exp2(-(y * y))
                g_ref[NR + t:NR + t + 1, t * N:(t + 1) * N] = mrow
            ec = (ei_row == c * TI + ei_col).astype(f32)
            xc = jnp.dot(xw1h, ec, preferred_element_type=f32) + b1h
            wcat = jnp.concatenate([w1rh, xc], axis=1)
            r = jnp.dot(wcat, g_ref[...], preferred_element_type=f32)
            for t in range(TI):
                hsum = hsum + _silu_from_half(r[:, t * N:(t + 1) * N])
            return hsum

        hsum = jax.lax.fori_loop(0, NCH, chunk, jnp.zeros((H, N), f32),
                                 unroll=2)

        aggr = jnp.dot(w2[...], hsum, preferred_element_type=f32) + b2[...] * cnt
        u = jnp.concatenate([xT, aggr], axis=0)
        hu = _silu_from_half(
            jnp.dot(u1h, u, preferred_element_type=f32) + ub1h)
        xT = xT + jnp.dot(u2[...], hu, preferred_element_type=f32) + ub2[...]

    sel = (batc_ref[...] == jax.lax.broadcasted_iota(jnp.int32, (1, NM), 1)).astype(f32)
    mol = jnp.dot(xT, sel, preferred_element_type=f32)
    cntm = jnp.sum(sel, axis=0, keepdims=True)
    mol = mol / jnp.clip(cntm, 1.0, None)
    ho = _silu_from_half(
        jnp.dot(ow1_ref[...] * 0.5, mol, preferred_element_type=f32)
        + ob1_ref[...] * 0.5)
    y_ref[...] = jnp.dot(ow2_ref[...], ho, preferred_element_type=f32) + ob2_ref[...]


def kernel(atomic_numbers, positions, batch, emb, blocks, out_w1, out_b1, out_w2, out_b2):
    f32 = jnp.float32
    anT = jnp.asarray(atomic_numbers, jnp.int32).reshape(1, N)
    posr = jnp.asarray(positions, f32)
    posT = posr.T
    batc = jnp.asarray(batch, jnp.int32).reshape(N, 1)
    embT = jnp.asarray(emb, f32).T
    wflat = []
    for blk in blocks:
        wflat += [
            blk['msg_w1'][:H].T, blk['msg_w1'][H:].T, blk['msg_b1'].reshape(H, 1),
            blk['msg_w2'].T, blk['msg_b2'].reshape(H, 1),
            blk['upd_w1'].T, blk['upd_b1'].reshape(H, 1),
            blk['upd_w2'].T, blk['upd_b2'].reshape(H, 1),
        ]
    yT = pl.pallas_call(
        _body,
        out_shape=jax.ShapeDtypeStruct((1, NM), f32),
        scratch_shapes=[pltpu.VMEM((KG, E), f32),
                        pltpu.VMEM((N, N), f32),
                        pltpu.VMEM((N, N), f32)],
    )(anT, posr, posT, batc, embT, *wflat,
      out_w1.T, out_b1.reshape(H // 2, 1), out_w2.T, out_b2.reshape(1, 1))
    return yT.reshape(NM, 1)
